# Initial kernel scaffold; baseline (speedup 1.0000x reference)
import jax
import jax.numpy as jnp
from jax.experimental import pallas as pl


def kernel(x, edge_index, batch, W1, b1, W2, b2, W3, b3):
    raise NotImplementedError("write your pallas kernel here")


# speedup vs baseline: 8.0254x; 8.0254x over previous
import functools

import jax
import jax.numpy as jnp
from jax import lax
from jax.experimental import pallas as pl
from jax.experimental.pallas import tpu as pltpu
from jax.experimental.pallas import tpu_sc as plsc

N = 10000
E = 320000
D = 128
G = 64

NC = 2
NS = 16
NW = NC * NS
K = 128
STEPS = (E + NW * K - 1) // (NW * K)
EPAD = NW * K * STEPS
ROWS_PER_TILE = 632
ACC_ROWS = NS * ROWS_PER_TILE
OUT_PER_TILE = ROWS_PER_TILE
RB = 10
BN = N // RB


def _fill(ref, rows, width, value):
    v = jnp.full((16,), value, jnp.float32)

    @pl.loop(0, rows)
    def _(i):
        for j in range(width // 16):
            ref[i, pl.ds(j * 16, 16)] = v


def _zero_acc_slice(zbuf, acc_sh, s, rows_per_tile, zrows):
    base = s * rows_per_tile
    off = 0
    while off < rows_per_tile:
        ch = min(zrows, rows_per_tile - off)
        pltpu.sync_copy(zbuf.at[pl.ds(0, ch)], acc_sh.at[pl.ds(base + off, ch)])
        off += ch


def _agg_body(g_hbm, src_hbm, dst_hbm, out_hbm,
              src_v, dst_v, rows0, acc_sh, sem0):
    c = lax.axis_index("c")
    s = lax.axis_index("s")
    w = s * NC + c
    pltpu.sync_copy(src_hbm.at[w], src_v)
    pltpu.sync_copy(dst_hbm.at[w], dst_v)
    _fill(rows0, K, D, 0.0)
    _zero_acc_slice(rows0, acc_sh, s, ROWS_PER_TILE, K)
    plsc.subcore_barrier()

    @pl.loop(0, STEPS)
    def _(j):
        pltpu.async_copy(g_hbm.at[src_v.at[j]], rows0, sem0).wait()
        pltpu.sync_copy(rows0, acc_sh.at[dst_v.at[j]], add=True)

    plsc.subcore_barrier()
    ob = s * OUT_PER_TILE
    pltpu.sync_copy(acc_sh.at[pl.ds(ob, OUT_PER_TILE)],
                    out_hbm.at[c, pl.ds(ob, OUT_PER_TILE)])


@functools.cache
def _agg_call():
    return pl.kernel(
        _agg_body,
        out_type=jax.ShapeDtypeStruct((NC, ACC_ROWS, D), jnp.float32),
        mesh=plsc.VectorSubcoreMesh(core_axis_name="c", subcore_axis_name="s"),
        scratch_types=[
            pltpu.VMEM((STEPS, K), jnp.int32),
            pltpu.VMEM((STEPS, K), jnp.int32),
            pltpu.VMEM((K, D), jnp.float32),
            pltpu.VMEM_SHARED((ACC_ROWS, D), jnp.float32),
            pltpu.SemaphoreType.DMA,
        ],
    )


def _k1_body(x_ref, w_ref, d0_ref, d1_ref, g_ref, dinv_ref, cs_ref):
    i = pl.program_id(0)
    x = x_ref[...]

    @pl.when(i == 0)
    def _():
        cs_ref[...] = jnp.zeros_like(cs_ref)

    cs_ref[...] += jnp.sum(x, axis=0, keepdims=True)
    dinv = lax.rsqrt(d0_ref[...] + d1_ref[...] + 1.0)
    dinv_ref[...] = dinv
    g_ref[...] = dinv * jnp.dot(x, w_ref[...],
                                preferred_element_type=jnp.float32)


def _kmid_body(a0_ref, a1_ref, gp_ref, dv_ref, b_ref, w_ref, out_ref):
    dinv = dv_ref[...]
    h = jnp.maximum(
        dinv * (a0_ref[...] + a1_ref[...] + gp_ref[...]) + b_ref[...], 0.0)
    out_ref[...] = dinv * jnp.dot(h, w_ref[...],
                                  preferred_element_type=jnp.float32)


def _k4_body(a0_ref, a1_ref, gp_ref, dv_ref, b_ref, batch_ref, cs_ref,
             pooled_ref, hc_ref, cnt_ref):
    i = pl.program_id(0)
    dinv = dv_ref[...]
    h = jnp.maximum(
        dinv * (a0_ref[...] + a1_ref[...] + gp_ref[...]) + b_ref[...], 0.0)
    seg = batch_ref[...]
    onehot = (seg == lax.broadcasted_iota(jnp.int32, (BN, G), 1)
              ).astype(jnp.float32)
    psum = lax.dot_general(onehot, h, (((0,), (0,)), ((), ())),
                           preferred_element_type=jnp.float32)
    ones_col = jnp.ones((BN, 1), jnp.float32)
    csum = lax.dot_general(onehot, ones_col, (((0,), (0,)), ((), ())),
                           preferred_element_type=jnp.float32)

    @pl.when(i == 0)
    def _():
        pooled_ref[...] = jnp.zeros_like(pooled_ref)
        cnt_ref[...] = jnp.zeros_like(cnt_ref)

    pooled_ref[...] += psum
    cnt_ref[...] += csum

    @pl.when(i == RB - 1)
    def _():
        pooled_ref[...] = pooled_ref[...] / jnp.maximum(cnt_ref[...], 1.0)
        cs = cs_ref[...]
        hc_ref[...] = cs / jnp.sum(cs)


_row_spec = pl.BlockSpec((BN, D), lambda i: (i, 0))
_col_spec = pl.BlockSpec((BN, 1), lambda i: (i, 0))
_w_spec = pl.BlockSpec((D, D), lambda i: (0, 0))
_vec_spec = pl.BlockSpec((1, D), lambda i: (0, 0))

_k1_call = pl.pallas_call(
    _k1_body,
    grid=(RB,),
    in_specs=[_row_spec, _w_spec, _col_spec, _col_spec],
    out_specs=[_row_spec, _col_spec, _vec_spec],
    out_shape=[
        jax.ShapeDtypeStruct((N, D), jnp.float32),
        jax.ShapeDtypeStruct((N, 1), jnp.float32),
        jax.ShapeDtypeStruct((1, D), jnp.float32),
    ],
)

_kmid_call = pl.pallas_call(
    _kmid_body,
    grid=(RB,),
    in_specs=[_row_spec, _row_spec, _row_spec, _col_spec, _vec_spec, _w_spec],
    out_specs=_row_spec,
    out_shape=jax.ShapeDtypeStruct((N, D), jnp.float32),
)

_k4_call = pl.pallas_call(
    _k4_body,
    grid=(RB,),
    in_specs=[_row_spec, _row_spec, _row_spec, _col_spec, _vec_spec,
              _col_spec, _vec_spec],
    out_specs=[pl.BlockSpec((G, D), lambda i: (0, 0)),
               pl.BlockSpec((1, D), lambda i: (0, 0))],
    out_shape=[
        jax.ShapeDtypeStruct((G, D), jnp.float32),
        jax.ShapeDtypeStruct((1, D), jnp.float32),
    ],
    scratch_shapes=[pltpu.VMEM((G, 1), jnp.float32)],
)


def kernel(x, edge_index, batch, W1, b1, W2, b2, W3, b3):
    src = edge_index[0]
    dst = edge_index[1]
    pad = EPAD - E
    srcp = jnp.concatenate(
        [src, jnp.zeros((pad,), jnp.int32)]).reshape(NW, STEPS, K)
    dstp = jnp.concatenate(
        [dst, jnp.full((pad,), N, jnp.int32)]).reshape(NW, STEPS, K)

    deg_p = _agg_call()(jnp.ones((N, D), jnp.float32), srcp, dstp)
    dp0 = deg_p[0, :N, 0:1]
    dp1 = deg_p[1, :N, 0:1]

    g1, dinv, colsum = _k1_call(x, W1, dp0, dp1)
    a1 = _agg_call()(g1, srcp, dstp)
    g2 = _kmid_call(a1[0, :N], a1[1, :N], g1, dinv, b1.reshape(1, D), W2)
    a2 = _agg_call()(g2, srcp, dstp)
    g3 = _kmid_call(a2[0, :N], a2[1, :N], g2, dinv, b2.reshape(1, D), W3)
    a3 = _agg_call()(g3, srcp, dstp)
    pooled, hc = _k4_call(a3[0, :N], a3[1, :N], g3, dinv, b3.reshape(1, D),
                          batch.reshape(N, 1), colsum)
    return (pooled, hc.reshape(D))


# speedup vs baseline: 10.7581x; 1.3405x over previous
import functools

import jax
import jax.numpy as jnp
from jax import lax
from jax.experimental import pallas as pl
from jax.experimental.pallas import tpu as pltpu
from jax.experimental.pallas import tpu_sc as plsc

N = 10000
E = 320000
D = 128
G = 64

NC = 2
NS = 16
NW = NC * NS
K = 128
STEPS = (E + NW * K - 1) // (NW * K)
EPAD = NW * K * STEPS
ROWS_PER_TILE = 632
ACC_ROWS = NS * ROWS_PER_TILE
OUT_PER_TILE = ROWS_PER_TILE
RB = 10
BN = N // RB


def _fill(ref, rows, width, value):
    v = jnp.full((16,), value, jnp.float32)

    @pl.loop(0, rows)
    def _(i):
        for j in range(width // 16):
            ref[i, pl.ds(j * 16, 16)] = v


def _zero_acc_slice(zbuf, acc_sh, s, rows_per_tile, zrows):
    base = s * rows_per_tile
    off = 0
    while off < rows_per_tile:
        ch = min(zrows, rows_per_tile - off)
        pltpu.sync_copy(zbuf.at[pl.ds(0, ch)], acc_sh.at[pl.ds(base + off, ch)])
        off += ch


def _agg_body(g_hbm, src_hbm, dst_hbm, out_hbm,
              src_i, dst_i, rows, acc_sh, sem_rows, sem_idx):
    c = lax.axis_index("c")
    s = lax.axis_index("s")
    w = s * NC + c
    _fill(rows, K, D, 0.0)
    _zero_acc_slice(rows, acc_sh, s, ROWS_PER_TILE, K)
    plsc.subcore_barrier()

    pltpu.sync_copy(src_hbm.at[w, 0], src_i.at[0])
    pltpu.sync_copy(dst_hbm.at[w, 0], dst_i.at[0])
    pltpu.sync_copy(src_hbm.at[w, 1], src_i.at[1])
    pltpu.sync_copy(dst_hbm.at[w, 1], dst_i.at[1])
    pltpu.async_copy(g_hbm.at[src_i.at[0]], rows.at[pl.ds(0, K)],
                     sem_rows.at[0])

    @pl.loop(0, STEPS)
    def _(j):
        ph = lax.rem(j, 2)
        nph = 1 - ph
        pltpu.make_async_copy(g_hbm.at[src_i.at[ph]],
                              rows.at[pl.ds(ph * K, K)],
                              sem_rows.at[ph]).wait()

        @pl.when(j + 1 < STEPS)
        def _():
            @pl.when(j >= 1)
            def _():
                pltpu.make_async_copy(src_hbm.at[w, j + 1], src_i.at[nph],
                                      sem_idx.at[nph]).wait()
                pltpu.make_async_copy(dst_hbm.at[w, j + 1], dst_i.at[nph],
                                      sem_idx.at[nph]).wait()
            pltpu.async_copy(g_hbm.at[src_i.at[nph]],
                             rows.at[pl.ds(nph * K, K)], sem_rows.at[nph])

        pltpu.sync_copy(rows.at[pl.ds(ph * K, K)],
                        acc_sh.at[dst_i.at[ph]], add=True)

        @pl.when(j + 2 < STEPS)
        def _():
            pltpu.async_copy(src_hbm.at[w, j + 2], src_i.at[ph],
                             sem_idx.at[ph])
            pltpu.async_copy(dst_hbm.at[w, j + 2], dst_i.at[ph],
                             sem_idx.at[ph])

    plsc.subcore_barrier()
    ob = s * OUT_PER_TILE
    pltpu.sync_copy(acc_sh.at[pl.ds(ob, OUT_PER_TILE)],
                    out_hbm.at[c, pl.ds(ob, OUT_PER_TILE)])


@functools.cache
def _agg_call():
    return pl.kernel(
        _agg_body,
        out_type=jax.ShapeDtypeStruct((NC, ACC_ROWS, D), jnp.float32),
        mesh=plsc.VectorSubcoreMesh(core_axis_name="c", subcore_axis_name="s"),
        scratch_types=[
            pltpu.VMEM((2, K), jnp.int32),
            pltpu.VMEM((2, K), jnp.int32),
            pltpu.VMEM((2 * K, D), jnp.float32),
            pltpu.VMEM_SHARED((ACC_ROWS, D), jnp.float32),
            pltpu.SemaphoreType.DMA((2,)),
            pltpu.SemaphoreType.DMA((2,)),
        ],
    )


def _deg_body(dst_hbm, out_hbm, dst_v, ones_v, acc_sh):
    c = lax.axis_index("c")
    s = lax.axis_index("s")
    w = s * NC + c
    pltpu.sync_copy(dst_hbm.at[w], dst_v)
    _fill(ones_v, K, D, 0.0)
    _zero_acc_slice(ones_v, acc_sh, s, ROWS_PER_TILE, K)
    plsc.subcore_barrier()
    _fill(ones_v, K, D, 1.0)

    @pl.loop(0, STEPS)
    def _(j):
        pltpu.sync_copy(ones_v, acc_sh.at[dst_v.at[j]], add=True)

    plsc.subcore_barrier()
    ob = s * OUT_PER_TILE
    pltpu.sync_copy(acc_sh.at[pl.ds(ob, OUT_PER_TILE)],
                    out_hbm.at[c, pl.ds(ob, OUT_PER_TILE)])


@functools.cache
def _deg_call():
    return pl.kernel(
        _deg_body,
        out_type=jax.ShapeDtypeStruct((NC, ACC_ROWS, D), jnp.float32),
        mesh=plsc.VectorSubcoreMesh(core_axis_name="c", subcore_axis_name="s"),
        scratch_types=[
            pltpu.VMEM((STEPS, K), jnp.int32),
            pltpu.VMEM((K, D), jnp.float32),
            pltpu.VMEM_SHARED((ACC_ROWS, D), jnp.float32),
        ],
    )


def _k1_body(x_ref, w_ref, d0_ref, d1_ref, g_ref, dinv_ref, cs_ref):
    i = pl.program_id(0)
    x = x_ref[...]

    @pl.when(i == 0)
    def _():
        cs_ref[...] = jnp.zeros_like(cs_ref)

    cs_ref[...] += jnp.sum(x, axis=0, keepdims=True)
    dinv = lax.rsqrt(d0_ref[...] + d1_ref[...] + 1.0)
    dinv_ref[...] = dinv
    g_ref[...] = dinv * jnp.dot(x, w_ref[...],
                                preferred_element_type=jnp.float32)


def _kmid_body(a0_ref, a1_ref, gp_ref, dv_ref, b_ref, w_ref, out_ref):
    dinv = dv_ref[...]
    h = jnp.maximum(
        dinv * (a0_ref[...] + a1_ref[...] + gp_ref[...]) + b_ref[...], 0.0)
    out_ref[...] = dinv * jnp.dot(h, w_ref[...],
                                  preferred_element_type=jnp.float32)


def _k4_body(a0_ref, a1_ref, gp_ref, dv_ref, b_ref, batch_ref, cs_ref,
             pooled_ref, hc_ref, cnt_ref):
    i = pl.program_id(0)
    dinv = dv_ref[...]
    h = jnp.maximum(
        dinv * (a0_ref[...] + a1_ref[...] + gp_ref[...]) + b_ref[...], 0.0)
    seg = batch_ref[...]
    onehot = (seg == lax.broadcasted_iota(jnp.int32, (BN, G), 1)
              ).astype(jnp.float32)
    psum = lax.dot_general(onehot, h, (((0,), (0,)), ((), ())),
                           preferred_element_type=jnp.float32)
    ones_col = jnp.ones((BN, 1), jnp.float32)
    csum = lax.dot_general(onehot, ones_col, (((0,), (0,)), ((), ())),
                           preferred_element_type=jnp.float32)

    @pl.when(i == 0)
    def _():
        pooled_ref[...] = jnp.zeros_like(pooled_ref)
        cnt_ref[...] = jnp.zeros_like(cnt_ref)

    pooled_ref[...] += psum
    cnt_ref[...] += csum

    @pl.when(i == RB - 1)
    def _():
        pooled_ref[...] = pooled_ref[...] / jnp.maximum(cnt_ref[...], 1.0)
        cs = cs_ref[...]
        hc_ref[...] = cs / jnp.sum(cs)


_row_spec = pl.BlockSpec((BN, D), lambda i: (i, 0))
_col_spec = pl.BlockSpec((BN, 1), lambda i: (i, 0))
_w_spec = pl.BlockSpec((D, D), lambda i: (0, 0))
_vec_spec = pl.BlockSpec((1, D), lambda i: (0, 0))

_k1_call = pl.pallas_call(
    _k1_body,
    grid=(RB,),
    in_specs=[_row_spec, _w_spec, _col_spec, _col_spec],
    out_specs=[_row_spec, _col_spec, _vec_spec],
    out_shape=[
        jax.ShapeDtypeStruct((N, D), jnp.float32),
        jax.ShapeDtypeStruct((N, 1), jnp.float32),
        jax.ShapeDtypeStruct((1, D), jnp.float32),
    ],
)

_kmid_call = pl.pallas_call(
    _kmid_body,
    grid=(RB,),
    in_specs=[_row_spec, _row_spec, _row_spec, _col_spec, _vec_spec, _w_spec],
    out_specs=_row_spec,
    out_shape=jax.ShapeDtypeStruct((N, D), jnp.float32),
)

_k4_call = pl.pallas_call(
    _k4_body,
    grid=(RB,),
    in_specs=[_row_spec, _row_spec, _row_spec, _col_spec, _vec_spec,
              _col_spec, _vec_spec],
    out_specs=[pl.BlockSpec((G, D), lambda i: (0, 0)),
               pl.BlockSpec((1, D), lambda i: (0, 0))],
    out_shape=[
        jax.ShapeDtypeStruct((G, D), jnp.float32),
        jax.ShapeDtypeStruct((1, D), jnp.float32),
    ],
    scratch_shapes=[pltpu.VMEM((G, 1), jnp.float32)],
)


def kernel(x, edge_index, batch, W1, b1, W2, b2, W3, b3):
    src = edge_index[0]
    dst = edge_index[1]
    pad = EPAD - E
    srcp = jnp.concatenate(
        [src, jnp.zeros((pad,), jnp.int32)]).reshape(NW, STEPS, K)
    dstp = jnp.concatenate(
        [dst, jnp.full((pad,), N, jnp.int32)]).reshape(NW, STEPS, K)

    deg_p = _deg_call()(dstp)
    dp0 = deg_p[0, :N, 0:1]
    dp1 = deg_p[1, :N, 0:1]

    g1, dinv, colsum = _k1_call(x, W1, dp0, dp1)
    a1 = _agg_call()(g1, srcp, dstp)
    g2 = _kmid_call(a1[0, :N], a1[1, :N], g1, dinv, b1.reshape(1, D), W2)
    a2 = _agg_call()(g2, srcp, dstp)
    g3 = _kmid_call(a2[0, :N], a2[1, :N], g2, dinv, b2.reshape(1, D), W3)
    a3 = _agg_call()(g3, srcp, dstp)
    pooled, hc = _k4_call(a3[0, :N], a3[1, :N], g3, dinv, b3.reshape(1, D),
                          batch.reshape(N, 1), colsum)
    return (pooled, hc.reshape(D))


# speedup vs baseline: 11.1391x; 1.0354x over previous
import functools

import jax
import jax.numpy as jnp
from jax import lax
from jax.experimental import pallas as pl
from jax.experimental.pallas import tpu as pltpu
from jax.experimental.pallas import tpu_sc as plsc

N = 10000
E = 320000
D = 128
G = 64

NC = 2
NS = 16
NW = NC * NS
K = 128
NBUF = 3
STEPS = (E + NW * K - 1) // (NW * K)
EPAD = NW * K * STEPS
ROWS_PER_TILE = 632
ACC_ROWS = NS * ROWS_PER_TILE
OUT_PER_TILE = ROWS_PER_TILE
RB = 10
BN = N // RB


def _fill(ref, rows, width, value):
    v = jnp.full((16,), value, jnp.float32)

    @pl.loop(0, rows)
    def _(i):
        for j in range(width // 16):
            ref[i, pl.ds(j * 16, 16)] = v


def _zero_acc_slice(zbuf, acc_sh, s, rows_per_tile, zrows):
    base = s * rows_per_tile
    off = 0
    while off < rows_per_tile:
        ch = min(zrows, rows_per_tile - off)
        pltpu.sync_copy(zbuf.at[pl.ds(0, ch)], acc_sh.at[pl.ds(base + off, ch)])
        off += ch


def _agg_body(g_hbm, src_hbm, dst_hbm, out_hbm,
              src_i, dst_i, rows, acc_sh, sem_rows, sem_idx):
    c = lax.axis_index("c")
    s = lax.axis_index("s")
    w = s * NC + c
    _fill(rows, K, D, 0.0)
    _zero_acc_slice(rows, acc_sh, s, ROWS_PER_TILE, K)
    plsc.subcore_barrier()

    for p in range(NBUF):
        pltpu.sync_copy(src_hbm.at[w, p], src_i.at[p])
        pltpu.sync_copy(dst_hbm.at[w, p], dst_i.at[p])
    pltpu.async_copy(g_hbm.at[src_i.at[0]], rows.at[pl.ds(0, K)],
                     sem_rows.at[0])
    pltpu.async_copy(g_hbm.at[src_i.at[1]], rows.at[pl.ds(K, K)],
                     sem_rows.at[1])

    @pl.loop(0, STEPS)
    def _(j):
        ph = lax.rem(j, NBUF)
        gph = lax.rem(j + 2, NBUF)
        pltpu.make_async_copy(g_hbm.at[src_i.at[ph]],
                              rows.at[pl.ds(ph * K, K)],
                              sem_rows.at[ph]).wait()

        @pl.when(j + 2 < STEPS)
        def _():
            @pl.when(j >= 1)
            def _():
                pltpu.make_async_copy(src_hbm.at[w, j + 2], src_i.at[gph],
                                      sem_idx.at[gph]).wait()
                pltpu.make_async_copy(dst_hbm.at[w, j + 2], dst_i.at[gph],
                                      sem_idx.at[gph]).wait()
            pltpu.async_copy(g_hbm.at[src_i.at[gph]],
                             rows.at[pl.ds(gph * K, K)], sem_rows.at[gph])

        pltpu.sync_copy(rows.at[pl.ds(ph * K, K)],
                        acc_sh.at[dst_i.at[ph]], add=True)

        @pl.when(j + 3 < STEPS)
        def _():
            pltpu.async_copy(src_hbm.at[w, j + 3], src_i.at[ph],
                             sem_idx.at[ph])
            pltpu.async_copy(dst_hbm.at[w, j + 3], dst_i.at[ph],
                             sem_idx.at[ph])

    plsc.subcore_barrier()
    ob = s * OUT_PER_TILE
    pltpu.sync_copy(acc_sh.at[pl.ds(ob, OUT_PER_TILE)],
                    out_hbm.at[c, pl.ds(ob, OUT_PER_TILE)])


@functools.cache
def _agg_call():
    return pl.kernel(
        _agg_body,
        out_type=jax.ShapeDtypeStruct((NC, ACC_ROWS, D), jnp.float32),
        mesh=plsc.VectorSubcoreMesh(core_axis_name="c", subcore_axis_name="s"),
        scratch_types=[
            pltpu.VMEM((NBUF, K), jnp.int32),
            pltpu.VMEM((NBUF, K), jnp.int32),
            pltpu.VMEM((NBUF * K, D), jnp.float32),
            pltpu.VMEM_SHARED((ACC_ROWS, D), jnp.float32),
            pltpu.SemaphoreType.DMA((NBUF,)),
            pltpu.SemaphoreType.DMA((NBUF,)),
        ],
    )


def _deg_body(dst_hbm, out_hbm, dst_v, ones_v, acc_sh):
    c = lax.axis_index("c")
    s = lax.axis_index("s")
    w = s * NC + c
    pltpu.sync_copy(dst_hbm.at[w], dst_v)
    _fill(ones_v, K, D, 0.0)
    _zero_acc_slice(ones_v, acc_sh, s, ROWS_PER_TILE, K)
    plsc.subcore_barrier()
    _fill(ones_v, K, D, 1.0)

    @pl.loop(0, STEPS)
    def _(j):
        pltpu.sync_copy(ones_v, acc_sh.at[dst_v.at[j]], add=True)

    plsc.subcore_barrier()
    ob = s * OUT_PER_TILE
    pltpu.sync_copy(acc_sh.at[pl.ds(ob, OUT_PER_TILE)],
                    out_hbm.at[c, pl.ds(ob, OUT_PER_TILE)])


@functools.cache
def _deg_call():
    return pl.kernel(
        _deg_body,
        out_type=jax.ShapeDtypeStruct((NC, ACC_ROWS, D), jnp.float32),
        mesh=plsc.VectorSubcoreMesh(core_axis_name="c", subcore_axis_name="s"),
        scratch_types=[
            pltpu.VMEM((STEPS, K), jnp.int32),
            pltpu.VMEM((K, D), jnp.float32),
            pltpu.VMEM_SHARED((ACC_ROWS, D), jnp.float32),
        ],
    )


def _k1_body(x_ref, w_ref, d0_ref, d1_ref, g_ref, dinv_ref, cs_ref):
    i = pl.program_id(0)
    x = x_ref[...]

    @pl.when(i == 0)
    def _():
        cs_ref[...] = jnp.zeros_like(cs_ref)

    cs_ref[...] += jnp.sum(x, axis=0, keepdims=True)
    dinv = lax.rsqrt(d0_ref[...] + d1_ref[...] + 1.0)
    dinv_ref[...] = dinv
    g_ref[...] = dinv * jnp.dot(x, w_ref[...],
                                preferred_element_type=jnp.float32)


def _kmid_body(a0_ref, a1_ref, gp_ref, dv_ref, b_ref, w_ref, out_ref):
    dinv = dv_ref[...]
    h = jnp.maximum(
        dinv * (a0_ref[...] + a1_ref[...] + gp_ref[...]) + b_ref[...], 0.0)
    out_ref[...] = dinv * jnp.dot(h, w_ref[...],
                                  preferred_element_type=jnp.float32)


def _k4_body(a0_ref, a1_ref, gp_ref, dv_ref, b_ref, batch_ref, cs_ref,
             pooled_ref, hc_ref, cnt_ref):
    i = pl.program_id(0)
    dinv = dv_ref[...]
    h = jnp.maximum(
        dinv * (a0_ref[...] + a1_ref[...] + gp_ref[...]) + b_ref[...], 0.0)
    seg = batch_ref[...]
    onehot = (seg == lax.broadcasted_iota(jnp.int32, (BN, G), 1)
              ).astype(jnp.float32)
    psum = lax.dot_general(onehot, h, (((0,), (0,)), ((), ())),
                           preferred_element_type=jnp.float32)
    ones_col = jnp.ones((BN, 1), jnp.float32)
    csum = lax.dot_general(onehot, ones_col, (((0,), (0,)), ((), ())),
                           preferred_element_type=jnp.float32)

    @pl.when(i == 0)
    def _():
        pooled_ref[...] = jnp.zeros_like(pooled_ref)
        cnt_ref[...] = jnp.zeros_like(cnt_ref)

    pooled_ref[...] += psum
    cnt_ref[...] += csum

    @pl.when(i == RB - 1)
    def _():
        pooled_ref[...] = pooled_ref[...] / jnp.maximum(cnt_ref[...], 1.0)
        cs = cs_ref[...]
        hc_ref[...] = cs / jnp.sum(cs)


_row_spec = pl.BlockSpec((BN, D), lambda i: (i, 0))
_col_spec = pl.BlockSpec((BN, 1), lambda i: (i, 0))
_w_spec = pl.BlockSpec((D, D), lambda i: (0, 0))
_vec_spec = pl.BlockSpec((1, D), lambda i: (0, 0))

_k1_call = pl.pallas_call(
    _k1_body,
    grid=(RB,),
    in_specs=[_row_spec, _w_spec, _col_spec, _col_spec],
    out_specs=[_row_spec, _col_spec, _vec_spec],
    out_shape=[
        jax.ShapeDtypeStruct((N, D), jnp.float32),
        jax.ShapeDtypeStruct((N, 1), jnp.float32),
        jax.ShapeDtypeStruct((1, D), jnp.float32),
    ],
)

_kmid_call = pl.pallas_call(
    _kmid_body,
    grid=(RB,),
    in_specs=[_row_spec, _row_spec, _row_spec, _col_spec, _vec_spec, _w_spec],
    out_specs=_row_spec,
    out_shape=jax.ShapeDtypeStruct((N, D), jnp.float32),
)

_k4_call = pl.pallas_call(
    _k4_body,
    grid=(RB,),
    in_specs=[_row_spec, _row_spec, _row_spec, _col_spec, _vec_spec,
              _col_spec, _vec_spec],
    out_specs=[pl.BlockSpec((G, D), lambda i: (0, 0)),
               pl.BlockSpec((1, D), lambda i: (0, 0))],
    out_shape=[
        jax.ShapeDtypeStruct((G, D), jnp.float32),
        jax.ShapeDtypeStruct((1, D), jnp.float32),
    ],
    scratch_shapes=[pltpu.VMEM((G, 1), jnp.float32)],
)


def kernel(x, edge_index, batch, W1, b1, W2, b2, W3, b3):
    src = edge_index[0]
    dst = edge_index[1]
    pad = EPAD - E
    srcp = jnp.concatenate(
        [src, jnp.zeros((pad,), jnp.int32)]).reshape(NW, STEPS, K)
    dstp = jnp.concatenate(
        [dst, jnp.full((pad,), N, jnp.int32)]).reshape(NW, STEPS, K)

    deg_p = _deg_call()(dstp)
    dp0 = deg_p[0, :N, 0:1]
    dp1 = deg_p[1, :N, 0:1]

    g1, dinv, colsum = _k1_call(x, W1, dp0, dp1)
    a1 = _agg_call()(g1, srcp, dstp)
    g2 = _kmid_call(a1[0, :N], a1[1, :N], g1, dinv, b1.reshape(1, D), W2)
    a2 = _agg_call()(g2, srcp, dstp)
    g3 = _kmid_call(a2[0, :N], a2[1, :N], g2, dinv, b2.reshape(1, D), W3)
    a3 = _agg_call()(g3, srcp, dstp)
    pooled, hc = _k4_call(a3[0, :N], a3[1, :N], g3, dinv, b3.reshape(1, D),
                          batch.reshape(N, 1), colsum)
    return (pooled, hc.reshape(D))


# speedup vs baseline: 11.2992x; 1.0144x over previous
import functools

import jax
import jax.numpy as jnp
from jax import lax
from jax.experimental import pallas as pl
from jax.experimental.pallas import tpu as pltpu
from jax.experimental.pallas import tpu_sc as plsc

N = 10000
E = 320000
D = 128
G = 64

NC = 2
NS = 16
NW = NC * NS
K = 128
NBUF = 3
STEPS = (E + NW * K - 1) // (NW * K)
EPAD = NW * K * STEPS
ROWS_PER_TILE = 632
ACC_ROWS = NS * ROWS_PER_TILE
OUT_PER_TILE = ROWS_PER_TILE
RB = 10
BN = N // RB


def _fill(ref, rows, width, value):
    v = jnp.full((16,), value, jnp.float32)

    @pl.loop(0, rows)
    def _(i):
        for j in range(width // 16):
            ref[i, pl.ds(j * 16, 16)] = v


def _zero_acc_slice(zbuf, acc_sh, s, rows_per_tile, zrows):
    base = s * rows_per_tile
    off = 0
    while off < rows_per_tile:
        ch = min(zrows, rows_per_tile - off)
        pltpu.sync_copy(zbuf.at[pl.ds(0, ch)], acc_sh.at[pl.ds(base + off, ch)])
        off += ch


def _agg_body(g_hbm, idx_hbm, out_hbm,
              idx_v, rows, acc_sh, sem_rows, sem_idx):
    c = lax.axis_index("c")
    s = lax.axis_index("s")
    w = s * NC + c
    _fill(rows, K, D, 0.0)
    _zero_acc_slice(rows, acc_sh, s, ROWS_PER_TILE, K)
    plsc.subcore_barrier()

    for p in range(NBUF):
        pltpu.sync_copy(idx_hbm.at[w, p], idx_v.at[p])
    pltpu.async_copy(g_hbm.at[idx_v.at[0, 0]], rows.at[pl.ds(0, K)],
                     sem_rows.at[0])
    pltpu.async_copy(g_hbm.at[idx_v.at[1, 0]], rows.at[pl.ds(K, K)],
                     sem_rows.at[1])

    @pl.loop(0, STEPS)
    def _(j):
        ph = lax.rem(j, NBUF)
        gph = lax.rem(j + 2, NBUF)
        pltpu.make_async_copy(g_hbm.at[idx_v.at[ph, 0]],
                              rows.at[pl.ds(ph * K, K)],
                              sem_rows.at[ph]).wait()

        @pl.when(j + 2 < STEPS)
        def _():
            @pl.when(j >= 1)
            def _():
                pltpu.make_async_copy(idx_hbm.at[w, j + 2], idx_v.at[gph],
                                      sem_idx.at[gph]).wait()
            pltpu.async_copy(g_hbm.at[idx_v.at[gph, 0]],
                             rows.at[pl.ds(gph * K, K)], sem_rows.at[gph])

        pltpu.sync_copy(rows.at[pl.ds(ph * K, K)],
                        acc_sh.at[idx_v.at[ph, 1]], add=True)

        @pl.when(j + 3 < STEPS)
        def _():
            pltpu.async_copy(idx_hbm.at[w, j + 3], idx_v.at[ph],
                             sem_idx.at[ph])

    plsc.subcore_barrier()
    ob = s * OUT_PER_TILE
    pltpu.sync_copy(acc_sh.at[pl.ds(ob, OUT_PER_TILE)],
                    out_hbm.at[c, pl.ds(ob, OUT_PER_TILE)])


@functools.cache
def _agg_call():
    return pl.kernel(
        _agg_body,
        out_type=jax.ShapeDtypeStruct((NC, ACC_ROWS, D), jnp.float32),
        mesh=plsc.VectorSubcoreMesh(core_axis_name="c", subcore_axis_name="s"),
        scratch_types=[
            pltpu.VMEM((NBUF, 2, K), jnp.int32),
            pltpu.VMEM((NBUF * K, D), jnp.float32),
            pltpu.VMEM_SHARED((ACC_ROWS, D), jnp.float32),
            pltpu.SemaphoreType.DMA((NBUF,)),
            pltpu.SemaphoreType.DMA((NBUF,)),
        ],
    )


def _deg_body(dst_hbm, out_hbm, dst_v, ones_v, acc_sh):
    c = lax.axis_index("c")
    s = lax.axis_index("s")
    w = s * NC + c
    pltpu.sync_copy(dst_hbm.at[w], dst_v)
    _fill(ones_v, K, D, 0.0)
    _zero_acc_slice(ones_v, acc_sh, s, ROWS_PER_TILE, K)
    plsc.subcore_barrier()
    _fill(ones_v, K, D, 1.0)

    @pl.loop(0, STEPS)
    def _(j):
        pltpu.sync_copy(ones_v, acc_sh.at[dst_v.at[j]], add=True)

    plsc.subcore_barrier()
    ob = s * OUT_PER_TILE
    pltpu.sync_copy(acc_sh.at[pl.ds(ob, OUT_PER_TILE)],
                    out_hbm.at[c, pl.ds(ob, OUT_PER_TILE)])


@functools.cache
def _deg_call():
    return pl.kernel(
        _deg_body,
        out_type=jax.ShapeDtypeStruct((NC, ACC_ROWS, D), jnp.float32),
        mesh=plsc.VectorSubcoreMesh(core_axis_name="c", subcore_axis_name="s"),
        scratch_types=[
            pltpu.VMEM((STEPS, K), jnp.int32),
            pltpu.VMEM((K, D), jnp.float32),
            pltpu.VMEM_SHARED((ACC_ROWS, D), jnp.float32),
        ],
    )


def _k1_body(x_ref, w_ref, d0_ref, d1_ref, g_ref, dinv_ref, cs_ref):
    i = pl.program_id(0)
    x = x_ref[...]

    @pl.when(i == 0)
    def _():
        cs_ref[...] = jnp.zeros_like(cs_ref)

    cs_ref[...] += jnp.sum(x, axis=0, keepdims=True)
    dinv = lax.rsqrt(d0_ref[...] + d1_ref[...] + 1.0)
    dinv_ref[...] = dinv
    g_ref[...] = dinv * jnp.dot(x, w_ref[...],
                                preferred_element_type=jnp.float32)


def _kmid_body(a0_ref, a1_ref, gp_ref, dv_ref, b_ref, w_ref, out_ref):
    dinv = dv_ref[...]
    h = jnp.maximum(
        dinv * (a0_ref[...] + a1_ref[...] + gp_ref[...]) + b_ref[...], 0.0)
    out_ref[...] = dinv * jnp.dot(h, w_ref[...],
                                  preferred_element_type=jnp.float32)


def _k4_body(a0_ref, a1_ref, gp_ref, dv_ref, b_ref, batch_ref, cs_ref,
             pooled_ref, hc_ref, cnt_ref):
    i = pl.program_id(0)
    dinv = dv_ref[...]
    h = jnp.maximum(
        dinv * (a0_ref[...] + a1_ref[...] + gp_ref[...]) + b_ref[...], 0.0)
    seg = batch_ref[...]
    onehot = (seg == lax.broadcasted_iota(jnp.int32, (BN, G), 1)
              ).astype(jnp.float32)
    psum = lax.dot_general(onehot, h, (((0,), (0,)), ((), ())),
                           preferred_element_type=jnp.float32)
    ones_col = jnp.ones((BN, 1), jnp.float32)
    csum = lax.dot_general(onehot, ones_col, (((0,), (0,)), ((), ())),
                           preferred_element_type=jnp.float32)

    @pl.when(i == 0)
    def _():
        pooled_ref[...] = jnp.zeros_like(pooled_ref)
        cnt_ref[...] = jnp.zeros_like(cnt_ref)

    pooled_ref[...] += psum
    cnt_ref[...] += csum

    @pl.when(i == RB - 1)
    def _():
        pooled_ref[...] = pooled_ref[...] / jnp.maximum(cnt_ref[...], 1.0)
        cs = cs_ref[...]
        hc_ref[...] = cs / jnp.sum(cs)


_row_spec = pl.BlockSpec((BN, D), lambda i: (i, 0))
_col_spec = pl.BlockSpec((BN, 1), lambda i: (i, 0))
_w_spec = pl.BlockSpec((D, D), lambda i: (0, 0))
_vec_spec = pl.BlockSpec((1, D), lambda i: (0, 0))

_k1_call = pl.pallas_call(
    _k1_body,
    grid=(RB,),
    in_specs=[_row_spec, _w_spec, _col_spec, _col_spec],
    out_specs=[_row_spec, _col_spec, _vec_spec],
    out_shape=[
        jax.ShapeDtypeStruct((N, D), jnp.float32),
        jax.ShapeDtypeStruct((N, 1), jnp.float32),
        jax.ShapeDtypeStruct((1, D), jnp.float32),
    ],
)

_kmid_call = pl.pallas_call(
    _kmid_body,
    grid=(RB,),
    in_specs=[_row_spec, _row_spec, _row_spec, _col_spec, _vec_spec, _w_spec],
    out_specs=_row_spec,
    out_shape=jax.ShapeDtypeStruct((N, D), jnp.float32),
)

_k4_call = pl.pallas_call(
    _k4_body,
    grid=(RB,),
    in_specs=[_row_spec, _row_spec, _row_spec, _col_spec, _vec_spec,
              _col_spec, _vec_spec],
    out_specs=[pl.BlockSpec((G, D), lambda i: (0, 0)),
               pl.BlockSpec((1, D), lambda i: (0, 0))],
    out_shape=[
        jax.ShapeDtypeStruct((G, D), jnp.float32),
        jax.ShapeDtypeStruct((1, D), jnp.float32),
    ],
    scratch_shapes=[pltpu.VMEM((G, 1), jnp.float32)],
)


def kernel(x, edge_index, batch, W1, b1, W2, b2, W3, b3):
    src = edge_index[0]
    dst = edge_index[1]
    pad = EPAD - E
    srcp = jnp.concatenate(
        [src, jnp.zeros((pad,), jnp.int32)]).reshape(NW, STEPS, K)
    dstp = jnp.concatenate(
        [dst, jnp.full((pad,), N, jnp.int32)]).reshape(NW, STEPS, K)
    idxp = jnp.stack([srcp, dstp], axis=2)

    deg_p = _deg_call()(dstp)
    dp0 = deg_p[0, :N, 0:1]
    dp1 = deg_p[1, :N, 0:1]

    g1, dinv, colsum = _k1_call(x, W1, dp0, dp1)
    a1 = _agg_call()(g1, idxp)
    g2 = _kmid_call(a1[0, :N], a1[1, :N], g1, dinv, b1.reshape(1, D), W2)
    a2 = _agg_call()(g2, idxp)
    g3 = _kmid_call(a2[0, :N], a2[1, :N], g2, dinv, b2.reshape(1, D), W3)
    a3 = _agg_call()(g3, idxp)
    pooled, hc = _k4_call(a3[0, :N], a3[1, :N], g3, dinv, b3.reshape(1, D),
                          batch.reshape(N, 1), colsum)
    return (pooled, hc.reshape(D))


# speedup vs baseline: 11.3008x; 1.0001x over previous
import functools

import jax
import jax.numpy as jnp
from jax import lax
from jax.experimental import pallas as pl
from jax.experimental.pallas import tpu as pltpu
from jax.experimental.pallas import tpu_sc as plsc

N = 10000
E = 320000
D = 128
G = 64

NC = 2
NS = 16
NW = NC * NS
K = 128
NBUF = 3
STEPS = (E + NW * K - 1) // (NW * K)
EPAD = NW * K * STEPS
ROWS_PER_TILE = 632
ACC_ROWS = NS * ROWS_PER_TILE
OUT_PER_TILE = ROWS_PER_TILE
RB = 10
BN = N // RB


def _fill(ref, rows, width, value):
    v = jnp.full((16,), value, jnp.float32)

    @pl.loop(0, rows)
    def _(i):
        for j in range(width // 16):
            ref[i, pl.ds(j * 16, 16)] = v


def _zero_acc_slice(zbuf, acc_sh, s, rows_per_tile, zrows):
    base = s * rows_per_tile
    off = 0
    while off < rows_per_tile:
        ch = min(zrows, rows_per_tile - off)
        pltpu.sync_copy(zbuf.at[pl.ds(0, ch)], acc_sh.at[pl.ds(base + off, ch)])
        off += ch


def _agg_body(g_hbm, idx_hbm, out_hbm,
              idx_v, rows, acc_sh, sem_rows, sem_idx):
    c = lax.axis_index("c")
    s = lax.axis_index("s")
    w = s * NC + c
    _fill(rows, K, D, 0.0)
    _zero_acc_slice(rows, acc_sh, s, ROWS_PER_TILE, K)
    plsc.subcore_barrier()

    for p in range(NBUF):
        pltpu.sync_copy(idx_hbm.at[w, p], idx_v.at[p])
    pltpu.async_copy(g_hbm.at[idx_v.at[0, 0]], rows.at[pl.ds(0, K)],
                     sem_rows.at[0])
    pltpu.async_copy(g_hbm.at[idx_v.at[1, 0]], rows.at[pl.ds(K, K)],
                     sem_rows.at[1])

    @pl.loop(0, STEPS)
    def _(j):
        ph = lax.rem(j, NBUF)
        gph = lax.rem(j + 2, NBUF)
        pltpu.make_async_copy(g_hbm.at[idx_v.at[ph, 0]],
                              rows.at[pl.ds(ph * K, K)],
                              sem_rows.at[ph]).wait()

        @pl.when(j + 2 < STEPS)
        def _():
            @pl.when(j >= 1)
            def _():
                pltpu.make_async_copy(idx_hbm.at[w, j + 2], idx_v.at[gph],
                                      sem_idx.at[gph]).wait()
            pltpu.async_copy(g_hbm.at[idx_v.at[gph, 0]],
                             rows.at[pl.ds(gph * K, K)], sem_rows.at[gph],
                             priority=1)

        pltpu.sync_copy(rows.at[pl.ds(ph * K, K)],
                        acc_sh.at[idx_v.at[ph, 1]], add=True)

        @pl.when(j + 3 < STEPS)
        def _():
            pltpu.async_copy(idx_hbm.at[w, j + 3], idx_v.at[ph],
                             sem_idx.at[ph])

    plsc.subcore_barrier()
    ob = s * OUT_PER_TILE
    pltpu.sync_copy(acc_sh.at[pl.ds(ob, OUT_PER_TILE)],
                    out_hbm.at[c, pl.ds(ob, OUT_PER_TILE)])


@functools.cache
def _agg_call():
    return pl.kernel(
        _agg_body,
        out_type=jax.ShapeDtypeStruct((NC, ACC_ROWS, D), jnp.float32),
        mesh=plsc.VectorSubcoreMesh(core_axis_name="c", subcore_axis_name="s"),
        scratch_types=[
            pltpu.VMEM((NBUF, 2, K), jnp.int32),
            pltpu.VMEM((NBUF * K, D), jnp.float32),
            pltpu.VMEM_SHARED((ACC_ROWS, D), jnp.float32),
            pltpu.SemaphoreType.DMA((NBUF,)),
            pltpu.SemaphoreType.DMA((NBUF,)),
        ],
    )


def _deg_body(dst_hbm, out_hbm, dst_v, ones_v, acc_sh):
    c = lax.axis_index("c")
    s = lax.axis_index("s")
    w = s * NC + c
    pltpu.sync_copy(dst_hbm.at[w], dst_v)
    _fill(ones_v, K, D, 0.0)
    _zero_acc_slice(ones_v, acc_sh, s, ROWS_PER_TILE, K)
    plsc.subcore_barrier()
    _fill(ones_v, K, D, 1.0)

    @pl.loop(0, STEPS)
    def _(j):
        pltpu.sync_copy(ones_v, acc_sh.at[dst_v.at[j]], add=True)

    plsc.subcore_barrier()
    ob = s * OUT_PER_TILE
    pltpu.sync_copy(acc_sh.at[pl.ds(ob, OUT_PER_TILE)],
                    out_hbm.at[c, pl.ds(ob, OUT_PER_TILE)])


@functools.cache
def _deg_call():
    return pl.kernel(
        _deg_body,
        out_type=jax.ShapeDtypeStruct((NC, ACC_ROWS, D), jnp.float32),
        mesh=plsc.VectorSubcoreMesh(core_axis_name="c", subcore_axis_name="s"),
        scratch_types=[
            pltpu.VMEM((STEPS, K), jnp.int32),
            pltpu.VMEM((K, D), jnp.float32),
            pltpu.VMEM_SHARED((ACC_ROWS, D), jnp.float32),
        ],
    )


def _k1_body(x_ref, w_ref, d0_ref, d1_ref, g_ref, dinv_ref, cs_ref):
    i = pl.program_id(0)
    x = x_ref[...]

    @pl.when(i == 0)
    def _():
        cs_ref[...] = jnp.zeros_like(cs_ref)

    cs_ref[...] += jnp.sum(x, axis=0, keepdims=True)
    dinv = lax.rsqrt(d0_ref[...] + d1_ref[...] + 1.0)
    dinv_ref[...] = dinv
    g_ref[...] = dinv * jnp.dot(x, w_ref[...],
                                preferred_element_type=jnp.float32)


def _kmid_body(a0_ref, a1_ref, gp_ref, dv_ref, b_ref, w_ref, out_ref):
    dinv = dv_ref[...]
    h = jnp.maximum(
        dinv * (a0_ref[...] + a1_ref[...] + gp_ref[...]) + b_ref[...], 0.0)
    out_ref[...] = dinv * jnp.dot(h, w_ref[...],
                                  preferred_element_type=jnp.float32)


def _k4_body(a0_ref, a1_ref, gp_ref, dv_ref, b_ref, batch_ref, cs_ref,
             pooled_ref, hc_ref, cnt_ref):
    i = pl.program_id(0)
    dinv = dv_ref[...]
    h = jnp.maximum(
        dinv * (a0_ref[...] + a1_ref[...] + gp_ref[...]) + b_ref[...], 0.0)
    seg = batch_ref[...]
    onehot = (seg == lax.broadcasted_iota(jnp.int32, (BN, G), 1)
              ).astype(jnp.float32)
    psum = lax.dot_general(onehot, h, (((0,), (0,)), ((), ())),
                           preferred_element_type=jnp.float32)
    ones_col = jnp.ones((BN, 1), jnp.float32)
    csum = lax.dot_general(onehot, ones_col, (((0,), (0,)), ((), ())),
                           preferred_element_type=jnp.float32)

    @pl.when(i == 0)
    def _():
        pooled_ref[...] = jnp.zeros_like(pooled_ref)
        cnt_ref[...] = jnp.zeros_like(cnt_ref)

    pooled_ref[...] += psum
    cnt_ref[...] += csum

    @pl.when(i == RB - 1)
    def _():
        pooled_ref[...] = pooled_ref[...] / jnp.maximum(cnt_ref[...], 1.0)
        cs = cs_ref[...]
        hc_ref[...] = cs / jnp.sum(cs)


_row_spec = pl.BlockSpec((BN, D), lambda i: (i, 0))
_col_spec = pl.BlockSpec((BN, 1), lambda i: (i, 0))
_w_spec = pl.BlockSpec((D, D), lambda i: (0, 0))
_vec_spec = pl.BlockSpec((1, D), lambda i: (0, 0))

_k1_call = pl.pallas_call(
    _k1_body,
    grid=(RB,),
    in_specs=[_row_spec, _w_spec, _col_spec, _col_spec],
    out_specs=[_row_spec, _col_spec, _vec_spec],
    out_shape=[
        jax.ShapeDtypeStruct((N, D), jnp.float32),
        jax.ShapeDtypeStruct((N, 1), jnp.float32),
        jax.ShapeDtypeStruct((1, D), jnp.float32),
    ],
)

_kmid_call = pl.pallas_call(
    _kmid_body,
    grid=(RB,),
    in_specs=[_row_spec, _row_spec, _row_spec, _col_spec, _vec_spec, _w_spec],
    out_specs=_row_spec,
    out_shape=jax.ShapeDtypeStruct((N, D), jnp.float32),
)

_k4_call = pl.pallas_call(
    _k4_body,
    grid=(RB,),
    in_specs=[_row_spec, _row_spec, _row_spec, _col_spec, _vec_spec,
              _col_spec, _vec_spec],
    out_specs=[pl.BlockSpec((G, D), lambda i: (0, 0)),
               pl.BlockSpec((1, D), lambda i: (0, 0))],
    out_shape=[
        jax.ShapeDtypeStruct((G, D), jnp.float32),
        jax.ShapeDtypeStruct((1, D), jnp.float32),
    ],
    scratch_shapes=[pltpu.VMEM((G, 1), jnp.float32)],
)


def kernel(x, edge_index, batch, W1, b1, W2, b2, W3, b3):
    src = edge_index[0]
    dst = edge_index[1]
    pad = EPAD - E
    srcp = jnp.concatenate(
        [src, jnp.zeros((pad,), jnp.int32)]).reshape(NW, STEPS, K)
    dstp = jnp.concatenate(
        [dst, jnp.full((pad,), N, jnp.int32)]).reshape(NW, STEPS, K)
    idxp = jnp.stack([srcp, dstp], axis=2)

    deg_p = _deg_call()(dstp)
    dp0 = deg_p[0, :N, 0:1]
    dp1 = deg_p[1, :N, 0:1]

    g1, dinv, colsum = _k1_call(x, W1, dp0, dp1)
    a1 = _agg_call()(g1, idxp)
    g2 = _kmid_call(a1[0, :N], a1[1, :N], g1, dinv, b1.reshape(1, D), W2)
    a2 = _agg_call()(g2, idxp)
    g3 = _kmid_call(a2[0, :N], a2[1, :N], g2, dinv, b2.reshape(1, D), W3)
    a3 = _agg_call()(g3, idxp)
    pooled, hc = _k4_call(a3[0, :N], a3[1, :N], g3, dinv, b3.reshape(1, D),
                          batch.reshape(N, 1), colsum)
    return (pooled, hc.reshape(D))
